# Initial kernel scaffold; baseline (speedup 1.0000x reference)
import jax
import jax.numpy as jnp
from jax.experimental import pallas as pl


def kernel(x, Ws1, Ws2, Ws3, W1, W2, W3, Wr):
    raise NotImplementedError("write your pallas kernel here")


# speedup vs baseline: 2.1210x; 2.1210x over previous
import functools

import jax
import jax.numpy as jnp
from jax.experimental import pallas as pl
from jax.experimental.pallas import tpu as pltpu

_E = 8
_I = 128
_SI = 256
_TB = 512


def _moe_body(x_ref, wa_ref, wb_ref, wc_ref, wr_ref, out_ref):
    xb = x_ref[...]
    xb_bf = xb.astype(jnp.bfloat16)

    logits = jnp.dot(xb, wr_ref[...], preferred_element_type=jnp.float32)
    probs = jax.nn.softmax(logits, axis=-1)
    idx = jax.lax.broadcasted_iota(jnp.int32, probs.shape, 1)
    m1 = jnp.max(probs, axis=-1, keepdims=True)
    i1 = jnp.min(jnp.where(probs == m1, idx, _E), axis=-1, keepdims=True)
    mask1 = idx == i1
    pm = jnp.where(mask1, -1.0, probs)
    m2 = jnp.max(pm, axis=-1, keepdims=True)
    i2 = jnp.min(jnp.where(pm == m2, idx, _E), axis=-1, keepdims=True)
    sel = mask1 | (idx == i2)
    w_full = jnp.where(sel, probs, 0.0) / (m1 + m2)

    ncols = _SI + _E * _I
    col = jax.lax.broadcasted_iota(jnp.int32, (_E, ncols), 1)
    row = jax.lax.broadcasted_iota(jnp.int32, (_E, ncols), 0)
    sel_mat = ((col < _SI) | ((col - _SI) // _I == row)).astype(jnp.float32)
    wexp = jnp.dot(w_full, sel_mat, preferred_element_type=jnp.float32)

    g = jnp.dot(xb_bf, wa_ref[...], preferred_element_type=jnp.float32)
    u = jnp.dot(xb_bf, wb_ref[...], preferred_element_type=jnp.float32)
    h = (g * jax.nn.sigmoid(g)) * u
    hw = (h * wexp).astype(jnp.bfloat16)
    out_ref[...] = jnp.dot(hw, wc_ref[...], preferred_element_type=jnp.float32)


@functools.partial(jax.jit, static_argnames=())
def kernel(x, Ws1, Ws2, Ws3, W1, W2, W3, Wr):
    B, T, C = x.shape
    ntok = B * T
    x_flat = x.reshape(ntok, C)
    wa = jnp.concatenate([Ws1, W1.transpose(1, 0, 2).reshape(C, _E * _I)], axis=1)
    wb = jnp.concatenate([Ws2, W2.transpose(1, 0, 2).reshape(C, _E * _I)], axis=1)
    wc = jnp.concatenate([Ws3, W3.reshape(_E * _I, C)], axis=0)
    wa = wa.astype(jnp.bfloat16)
    wb = wb.astype(jnp.bfloat16)
    wc = wc.astype(jnp.bfloat16)

    ncols = _SI + _E * _I
    grid = (ntok // _TB,)
    out = pl.pallas_call(
        _moe_body,
        grid=grid,
        in_specs=[
            pl.BlockSpec((_TB, C), lambda i: (i, 0)),
            pl.BlockSpec((C, ncols), lambda i: (0, 0)),
            pl.BlockSpec((C, ncols), lambda i: (0, 0)),
            pl.BlockSpec((ncols, C), lambda i: (0, 0)),
            pl.BlockSpec((C, _E), lambda i: (0, 0)),
        ],
        out_specs=pl.BlockSpec((_TB, C), lambda i: (i, 0)),
        out_shape=jax.ShapeDtypeStruct((ntok, C), jnp.float32),
        compiler_params=pltpu.CompilerParams(
            dimension_semantics=("parallel",),
        ),
    )(x_flat, wa, wb, wc, Wr)
    return out.reshape(B, T, C)


# speedup vs baseline: 2.1956x; 1.0352x over previous
import functools

import jax
import jax.numpy as jnp
from jax.experimental import pallas as pl
from jax.experimental.pallas import tpu as pltpu

_E = 8
_I = 128
_SI = 256
_TB = 512


def _moe_body(x_ref, wa_ref, wb_ref, wc_ref, wr2_ref, out_ref):
    xb = x_ref[...]
    xhi = xb.astype(jnp.bfloat16)
    xlo = (xb - xhi.astype(jnp.float32)).astype(jnp.bfloat16)

    wr2 = wr2_ref[...]
    l1 = jnp.dot(xhi, wr2, preferred_element_type=jnp.float32)
    l2 = jnp.dot(xlo, wr2[:, :_E], preferred_element_type=jnp.float32)
    logits = l1[:, :_E] + l1[:, _E:] + l2

    probs = jax.nn.softmax(logits, axis=-1)
    idx = jax.lax.broadcasted_iota(jnp.int32, probs.shape, 1)
    m1 = jnp.max(probs, axis=-1, keepdims=True)
    i1 = jnp.min(jnp.where(probs == m1, idx, _E), axis=-1, keepdims=True)
    mask1 = idx == i1
    pm = jnp.where(mask1, -1.0, probs)
    m2 = jnp.max(pm, axis=-1, keepdims=True)
    i2 = jnp.min(jnp.where(pm == m2, idx, _E), axis=-1, keepdims=True)
    sel = mask1 | (idx == i2)
    w_full = jnp.where(sel, probs, 0.0) / (m1 + m2)

    g = jnp.dot(xhi, wa_ref[...], preferred_element_type=jnp.float32)
    u = jnp.dot(xhi, wb_ref[...], preferred_element_type=jnp.float32)
    h = ((g * jax.nn.sigmoid(g)) * u).astype(jnp.bfloat16)

    acc = jnp.dot(h[:, :_SI], wc_ref[:_SI, :], preferred_element_type=jnp.float32)
    for e in range(_E):
        lo = _SI + e * _I
        ye = jnp.dot(h[:, lo:lo + _I], wc_ref[lo:lo + _I, :],
                     preferred_element_type=jnp.float32)
        acc += w_full[:, e:e + 1] * ye
    out_ref[...] = acc


@functools.partial(jax.jit, static_argnames=())
def kernel(x, Ws1, Ws2, Ws3, W1, W2, W3, Wr):
    B, T, C = x.shape
    ntok = B * T
    x_flat = x.reshape(ntok, C)
    wa = jnp.concatenate([Ws1, W1.transpose(1, 0, 2).reshape(C, _E * _I)], axis=1)
    wb = jnp.concatenate([Ws2, W2.transpose(1, 0, 2).reshape(C, _E * _I)], axis=1)
    wc = jnp.concatenate([Ws3, W3.reshape(_E * _I, C)], axis=0)
    wa = wa.astype(jnp.bfloat16)
    wb = wb.astype(jnp.bfloat16)
    wc = wc.astype(jnp.bfloat16)
    wr_hi = Wr.astype(jnp.bfloat16)
    wr_lo = (Wr - wr_hi.astype(jnp.float32)).astype(jnp.bfloat16)
    wr2 = jnp.concatenate([wr_hi, wr_lo], axis=1)

    ncols = _SI + _E * _I
    grid = (ntok // _TB,)
    out = pl.pallas_call(
        _moe_body,
        grid=grid,
        in_specs=[
            pl.BlockSpec((_TB, C), lambda i: (i, 0)),
            pl.BlockSpec((C, ncols), lambda i: (0, 0)),
            pl.BlockSpec((C, ncols), lambda i: (0, 0)),
            pl.BlockSpec((ncols, C), lambda i: (0, 0)),
            pl.BlockSpec((C, 2 * _E), lambda i: (0, 0)),
        ],
        out_specs=pl.BlockSpec((_TB, C), lambda i: (i, 0)),
        out_shape=jax.ShapeDtypeStruct((ntok, C), jnp.float32),
        compiler_params=pltpu.CompilerParams(
            dimension_semantics=("parallel",),
        ),
    )(x_flat, wa, wb, wc, wr2)
    return out.reshape(B, T, C)


# speedup vs baseline: 2.2268x; 1.0142x over previous
import functools

import jax
import jax.numpy as jnp
from jax.experimental import pallas as pl
from jax.experimental.pallas import tpu as pltpu

_E = 8
_I = 128
_SI = 256
_TB = 512


def _moe_body(x_ref, wa_ref, wb_ref, wc_ref, wr2_ref, out_ref):
    xb = x_ref[...]
    xhi = xb.astype(jnp.bfloat16)

    logits = jnp.dot(xb, wr2_ref[...], preferred_element_type=jnp.float32)

    probs = jax.nn.softmax(logits, axis=-1)
    idx = jax.lax.broadcasted_iota(jnp.int32, probs.shape, 1)
    m1 = jnp.max(probs, axis=-1, keepdims=True)
    i1 = jnp.min(jnp.where(probs == m1, idx, _E), axis=-1, keepdims=True)
    mask1 = idx == i1
    pm = jnp.where(mask1, -1.0, probs)
    m2 = jnp.max(pm, axis=-1, keepdims=True)
    i2 = jnp.min(jnp.where(pm == m2, idx, _E), axis=-1, keepdims=True)
    sel = mask1 | (idx == i2)
    w_full = jnp.where(sel, probs, 0.0) / (m1 + m2)

    g = jnp.dot(xhi, wa_ref[...], preferred_element_type=jnp.float32)
    u = jnp.dot(xhi, wb_ref[...], preferred_element_type=jnp.float32)
    h = ((g * jax.nn.sigmoid(g)) * u).astype(jnp.bfloat16)

    acc = jnp.dot(h[:, :_SI], wc_ref[:_SI, :], preferred_element_type=jnp.float32)
    for e in range(_E):
        lo = _SI + e * _I
        ye = jnp.dot(h[:, lo:lo + _I], wc_ref[lo:lo + _I, :],
                     preferred_element_type=jnp.float32)
        acc += w_full[:, e:e + 1] * ye
    out_ref[...] = acc


@functools.partial(jax.jit, static_argnames=())
def kernel(x, Ws1, Ws2, Ws3, W1, W2, W3, Wr):
    B, T, C = x.shape
    ntok = B * T
    x_flat = x.reshape(ntok, C)
    wa = jnp.concatenate([Ws1, W1.transpose(1, 0, 2).reshape(C, _E * _I)], axis=1)
    wb = jnp.concatenate([Ws2, W2.transpose(1, 0, 2).reshape(C, _E * _I)], axis=1)
    wc = jnp.concatenate([Ws3, W3.reshape(_E * _I, C)], axis=0)
    wa = wa.astype(jnp.bfloat16)
    wb = wb.astype(jnp.bfloat16)
    wc = wc.astype(jnp.bfloat16)
    ncols = _SI + _E * _I
    grid = (ntok // _TB,)
    out = pl.pallas_call(
        _moe_body,
        grid=grid,
        in_specs=[
            pl.BlockSpec((_TB, C), lambda i: (i, 0)),
            pl.BlockSpec((C, ncols), lambda i: (0, 0)),
            pl.BlockSpec((C, ncols), lambda i: (0, 0)),
            pl.BlockSpec((ncols, C), lambda i: (0, 0)),
            pl.BlockSpec((C, _E), lambda i: (0, 0)),
        ],
        out_specs=pl.BlockSpec((_TB, C), lambda i: (i, 0)),
        out_shape=jax.ShapeDtypeStruct((ntok, C), jnp.float32),
        compiler_params=pltpu.CompilerParams(
            dimension_semantics=("parallel",),
        ),
    )(x_flat, wa, wb, wc, Wr)
    return out.reshape(B, T, C)


# speedup vs baseline: 2.6603x; 1.1947x over previous
import functools

import jax
import jax.numpy as jnp
from jax.experimental import pallas as pl
from jax.experimental.pallas import tpu as pltpu

_E = 8
_I = 128
_SI = 256
_TB = 512


def _moe_body(x_ref, wab_ref, wc_ref, wr_ref, out_ref):
    ncols = _SI + _E * _I
    xb = x_ref[...]
    xhi = xb.astype(jnp.bfloat16)

    logits = jnp.dot(xb, wr_ref[...], preferred_element_type=jnp.float32)

    idx = jax.lax.broadcasted_iota(jnp.int32, logits.shape, 1)
    m1 = jnp.max(logits, axis=-1, keepdims=True)
    i1 = jnp.min(jnp.where(logits == m1, idx, _E), axis=-1, keepdims=True)
    mask1 = idx == i1
    lm = jnp.where(mask1, -jnp.inf, logits)
    m2 = jnp.max(lm, axis=-1, keepdims=True)
    i2 = jnp.min(jnp.where(lm == m2, idx, _E), axis=-1, keepdims=True)
    mask2 = idx == i2
    d = jnp.exp(m2 - m1)
    w1 = 1.0 / (1.0 + d)
    w2 = 1.0 - w1

    gu = jnp.dot(xhi, wab_ref[...], preferred_element_type=jnp.float32)
    g = gu[:, :ncols]
    u = gu[:, ncols:]
    h = (g * jax.nn.sigmoid(g)) * u

    wcols = [jnp.ones((h.shape[0], _SI), jnp.float32)]
    for e in range(_E):
        we = jnp.where(mask1[:, e:e + 1], w1, 0.0) + \
             jnp.where(mask2[:, e:e + 1], w2, 0.0)
        wcols.append(jnp.broadcast_to(we, (h.shape[0], _I)))
    wexp = jnp.concatenate(wcols, axis=1)

    hw = (h * wexp).astype(jnp.bfloat16)
    out_ref[...] = jnp.dot(hw, wc_ref[...], preferred_element_type=jnp.float32)


@functools.partial(jax.jit, static_argnames=())
def kernel(x, Ws1, Ws2, Ws3, W1, W2, W3, Wr):
    B, T, C = x.shape
    ntok = B * T
    x_flat = x.reshape(ntok, C)
    wa = jnp.concatenate([Ws1, W1.transpose(1, 0, 2).reshape(C, _E * _I)], axis=1)
    wb = jnp.concatenate([Ws2, W2.transpose(1, 0, 2).reshape(C, _E * _I)], axis=1)
    wab = jnp.concatenate([wa, wb], axis=1).astype(jnp.bfloat16)
    wc = jnp.concatenate([Ws3, W3.reshape(_E * _I, C)], axis=0).astype(jnp.bfloat16)
    ncols = _SI + _E * _I
    grid = (ntok // _TB,)
    out = pl.pallas_call(
        _moe_body,
        grid=grid,
        in_specs=[
            pl.BlockSpec((_TB, C), lambda i: (i, 0)),
            pl.BlockSpec((C, 2 * ncols), lambda i: (0, 0)),
            pl.BlockSpec((ncols, C), lambda i: (0, 0)),
            pl.BlockSpec((C, _E), lambda i: (0, 0)),
        ],
        out_specs=pl.BlockSpec((_TB, C), lambda i: (i, 0)),
        out_shape=jax.ShapeDtypeStruct((ntok, C), jnp.float32),
        compiler_params=pltpu.CompilerParams(
            dimension_semantics=("parallel",),
        ),
    )(x_flat, wab, wc, Wr)
    return out.reshape(B, T, C)
